# Initial kernel scaffold; baseline (speedup 1.0000x reference)
import jax
import jax.numpy as jnp
from jax.experimental import pallas as pl


def kernel(msg, x_i, x_j, e_ij, index, num_nodes, W1, b1, W2, b2):
    raise NotImplementedError("write your pallas kernel here")


# speedup vs baseline: 1.1453x; 1.1453x over previous
import functools

import jax
import jax.numpy as jnp
from jax import lax
from jax.experimental import pallas as pl
from jax.experimental.pallas import tpu as pltpu
import jax.experimental.pallas.tpu_sc as plsc

E = 320000
N = 10000
D = 128
DE = 16
HIDDEN = 128

NC = 2
NS = 16
CHUNK = 128
CHUNKS_PER_W = 80
EDGES_PER_W = CHUNK * CHUNKS_PER_W
E_PAD = NC * NS * EDGES_PER_W
ROWS_PER_TILE = 632
N_PAD = NS * ROWS_PER_TILE

BLK = 256


def _mlp_body(msg_ref, xi_ref, xj_ref, e_ref, w1a_ref, w1b_ref, w1c_ref,
              b1_ref, w2_ref, b2_ref, out_ref):
    h = jnp.dot(xj_ref[...], w1a_ref[...], preferred_element_type=jnp.float32)
    h = h + jnp.dot(e_ref[...], w1b_ref[...], preferred_element_type=jnp.float32)
    h = h + jnp.dot(xi_ref[...], w1c_ref[...], preferred_element_type=jnp.float32)
    h = h + b1_ref[...]
    h = jnp.where(h >= 0, h, 0.01 * h)
    w = jnp.sum(h * w2_ref[...], axis=1, keepdims=True) + b2_ref[0, 0]
    w = jnp.tanh(w)
    out_ref[...] = w * msg_ref[...]


def _gated_mlp(msg, x_i, x_j, e_ij, W1, b1, W2, b2):
    w1a = W1[:D]
    w1b = W1[D:D + DE]
    w1c = W1[D + DE:]
    b1r = b1.reshape(1, HIDDEN)
    w2r = W2.reshape(1, HIDDEN)
    b2r = b2.reshape(1, 1)
    grid = E // BLK
    return pl.pallas_call(
        _mlp_body,
        grid=(grid,),
        in_specs=[
            pl.BlockSpec((BLK, D), lambda i: (i, 0)),
            pl.BlockSpec((BLK, D), lambda i: (i, 0)),
            pl.BlockSpec((BLK, D), lambda i: (i, 0)),
            pl.BlockSpec((BLK, DE), lambda i: (i, 0)),
            pl.BlockSpec((D, HIDDEN), lambda i: (0, 0)),
            pl.BlockSpec((DE, HIDDEN), lambda i: (0, 0)),
            pl.BlockSpec((D, HIDDEN), lambda i: (0, 0)),
            pl.BlockSpec((1, HIDDEN), lambda i: (0, 0)),
            pl.BlockSpec((1, HIDDEN), lambda i: (0, 0)),
            pl.BlockSpec((1, 1), lambda i: (0, 0)),
        ],
        out_specs=pl.BlockSpec((BLK, D), lambda i: (i, 0)),
        out_shape=jax.ShapeDtypeStruct((E_PAD, D), jnp.float32),
    )(msg, x_i, x_j, e_ij, w1a, w1b, w1c, b1r, w2r, b2r)


def _sc_scatter_body(gated_hbm, idx_hbm, zrows_hbm, out_hbm,
                     idxbuf, gbuf, accum):
    c = lax.axis_index("c")
    s = lax.axis_index("s")
    w = c * NS + s
    pltpu.sync_copy(idx_hbm.at[pl.ds(w * CHUNKS_PER_W, CHUNKS_PER_W)], idxbuf)
    pltpu.sync_copy(zrows_hbm, accum.at[pl.ds(s * ROWS_PER_TILE, ROWS_PER_TILE)])
    plsc.subcore_barrier()

    row0 = w * EDGES_PER_W

    def chunk(j, carry):
        pltpu.sync_copy(gated_hbm.at[pl.ds(row0 + j * CHUNK, CHUNK)], gbuf)
        pltpu.sync_copy(gbuf, accum.at[idxbuf.at[j]], add=True)
        return carry

    lax.fori_loop(0, CHUNKS_PER_W, chunk, 0)
    plsc.subcore_barrier()
    pltpu.sync_copy(accum.at[pl.ds(s * ROWS_PER_TILE, ROWS_PER_TILE)],
                    out_hbm.at[c, pl.ds(s * ROWS_PER_TILE, ROWS_PER_TILE)])


_sc_scatter = functools.partial(
    pl.kernel,
    out_type=jax.ShapeDtypeStruct((NC, N_PAD, D), jnp.float32),
    mesh=plsc.VectorSubcoreMesh(core_axis_name="c", subcore_axis_name="s"),
    scratch_types=[
        pltpu.VMEM((CHUNKS_PER_W, CHUNK), jnp.int32),
        pltpu.VMEM((CHUNK, D), jnp.float32),
        pltpu.VMEM_SHARED((N_PAD, D), jnp.float32),
    ],
)(_sc_scatter_body)


CBLK = 2000


def _combine_body(a_ref, b_ref, out_ref):
    out_ref[...] = a_ref[...] + b_ref[...]


def _combine(partials):
    return pl.pallas_call(
        _combine_body,
        grid=(N // CBLK,),
        in_specs=[
            pl.BlockSpec((None, CBLK, D), lambda i: (0, i, 0)),
            pl.BlockSpec((None, CBLK, D), lambda i: (1, i, 0)),
        ],
        out_specs=pl.BlockSpec((CBLK, D), lambda i: (i, 0)),
        out_shape=jax.ShapeDtypeStruct((N, D), jnp.float32),
    )(partials, partials)


def kernel(msg, x_i, x_j, e_ij, index, num_nodes, W1, b1, W2, b2):
    gated = _gated_mlp(msg, x_i, x_j, e_ij, W1, b1, W2, b2)
    idx = index.astype(jnp.int32)
    idx_pad = jnp.concatenate(
        [idx, jnp.full((E_PAD - E,), N, jnp.int32)]).reshape(-1, CHUNK)
    zrows = jnp.zeros((ROWS_PER_TILE, D), jnp.float32)
    partials = _sc_scatter(gated, idx_pad, zrows)
    return _combine(partials)


# speedup vs baseline: 2.0782x; 1.8146x over previous
import functools

import jax
import jax.numpy as jnp
from jax import lax
from jax.experimental import pallas as pl
from jax.experimental.pallas import tpu as pltpu
import jax.experimental.pallas.tpu_sc as plsc

E = 320000
N = 10000
D = 128
DE = 16
HIDDEN = 128

NC = 2
NS = 16
CHUNK = 128
CHUNKS_PER_W = 80
EDGES_PER_W = CHUNK * CHUNKS_PER_W
E_PAD = NC * NS * EDGES_PER_W
ROWS_PER_TILE = 632
N_PAD = NS * ROWS_PER_TILE

BLK = 1024


def _mlp_body(msg_ref, xi_ref, xj_ref, e_ref, w1a_ref, w1b_ref, w1c_ref,
              b1_ref, w2_ref, b2_ref, out_ref):
    h = jnp.dot(xj_ref[...], w1a_ref[...], preferred_element_type=jnp.float32)
    h = h + jnp.dot(e_ref[...], w1b_ref[...], preferred_element_type=jnp.float32)
    h = h + jnp.dot(xi_ref[...], w1c_ref[...], preferred_element_type=jnp.float32)
    h = h + b1_ref[...]
    h = jnp.where(h >= 0, h, 0.01 * h)
    w = jnp.sum(h * w2_ref[...], axis=1, keepdims=True) + b2_ref[0, 0]
    w = jnp.tanh(w)
    out_ref[...] = w * msg_ref[...]


def _gated_mlp(msg, x_i, x_j, e_ij, W1, b1, W2, b2):
    w1a = W1[:D]
    w1b = W1[D:D + DE]
    w1c = W1[D + DE:]
    b1r = b1.reshape(1, HIDDEN)
    w2r = W2.reshape(1, HIDDEN)
    b2r = b2.reshape(1, 1)
    grid = E // BLK
    return pl.pallas_call(
        _mlp_body,
        grid=(grid,),
        in_specs=[
            pl.BlockSpec((BLK, D), lambda i: (i, 0)),
            pl.BlockSpec((BLK, D), lambda i: (i, 0)),
            pl.BlockSpec((BLK, D), lambda i: (i, 0)),
            pl.BlockSpec((BLK, DE), lambda i: (i, 0)),
            pl.BlockSpec((D, HIDDEN), lambda i: (0, 0)),
            pl.BlockSpec((DE, HIDDEN), lambda i: (0, 0)),
            pl.BlockSpec((D, HIDDEN), lambda i: (0, 0)),
            pl.BlockSpec((1, HIDDEN), lambda i: (0, 0)),
            pl.BlockSpec((1, HIDDEN), lambda i: (0, 0)),
            pl.BlockSpec((1, 1), lambda i: (0, 0)),
        ],
        out_specs=pl.BlockSpec((BLK, D), lambda i: (i, 0)),
        out_shape=jax.ShapeDtypeStruct((E_PAD, D), jnp.float32),
    )(msg, x_i, x_j, e_ij, w1a, w1b, w1c, b1r, w2r, b2r)


def _sc_scatter_body(gated_hbm, idx_hbm, zrows_hbm, out_hbm,
                     idxbuf, gbuf, accum):
    c = lax.axis_index("c")
    s = lax.axis_index("s")
    w = c * NS + s
    pltpu.sync_copy(idx_hbm.at[pl.ds(w * CHUNKS_PER_W, CHUNKS_PER_W)], idxbuf)
    pltpu.sync_copy(zrows_hbm, accum.at[pl.ds(s * ROWS_PER_TILE, ROWS_PER_TILE)])
    plsc.subcore_barrier()

    row0 = w * EDGES_PER_W

    def chunk(j, carry):
        pltpu.sync_copy(gated_hbm.at[pl.ds(row0 + j * CHUNK, CHUNK)], gbuf)
        pltpu.sync_copy(gbuf, accum.at[idxbuf.at[j]], add=True)
        return carry

    lax.fori_loop(0, CHUNKS_PER_W, chunk, 0)
    plsc.subcore_barrier()
    pltpu.sync_copy(accum.at[pl.ds(s * ROWS_PER_TILE, ROWS_PER_TILE)],
                    out_hbm.at[c, pl.ds(s * ROWS_PER_TILE, ROWS_PER_TILE)])


_sc_scatter = functools.partial(
    pl.kernel,
    out_type=jax.ShapeDtypeStruct((NC, N_PAD, D), jnp.float32),
    mesh=plsc.VectorSubcoreMesh(core_axis_name="c", subcore_axis_name="s"),
    scratch_types=[
        pltpu.VMEM((CHUNKS_PER_W, CHUNK), jnp.int32),
        pltpu.VMEM((CHUNK, D), jnp.float32),
        pltpu.VMEM_SHARED((N_PAD, D), jnp.float32),
    ],
)(_sc_scatter_body)


CBLK = 2000


def _combine_body(a_ref, b_ref, out_ref):
    out_ref[...] = a_ref[...] + b_ref[...]


def _combine(partials):
    return pl.pallas_call(
        _combine_body,
        grid=(N // CBLK,),
        in_specs=[
            pl.BlockSpec((None, CBLK, D), lambda i: (0, i, 0)),
            pl.BlockSpec((None, CBLK, D), lambda i: (1, i, 0)),
        ],
        out_specs=pl.BlockSpec((CBLK, D), lambda i: (i, 0)),
        out_shape=jax.ShapeDtypeStruct((N, D), jnp.float32),
    )(partials, partials)


def kernel(msg, x_i, x_j, e_ij, index, num_nodes, W1, b1, W2, b2):
    gated = _gated_mlp(msg, x_i, x_j, e_ij, W1, b1, W2, b2)
    idx = index.astype(jnp.int32)
    idx_pad = jnp.concatenate(
        [idx, jnp.full((E_PAD - E,), N, jnp.int32)]).reshape(-1, CHUNK)
    zrows = jnp.zeros((ROWS_PER_TILE, D), jnp.float32)
    partials = _sc_scatter(gated, idx_pad, zrows)
    return _combine(partials)


# speedup vs baseline: 2.5379x; 1.2212x over previous
import functools

import jax
import jax.numpy as jnp
from jax import lax
from jax.experimental import pallas as pl
from jax.experimental.pallas import tpu as pltpu
import jax.experimental.pallas.tpu_sc as plsc

E = 320000
N = 10000
D = 128
DE = 16
HIDDEN = 128

NC = 2
NS = 16
CHUNK = 128
CHUNKS_PER_W = 80
EDGES_PER_W = CHUNK * CHUNKS_PER_W
E_PAD = NC * NS * EDGES_PER_W
ROWS_PER_TILE = 632
N_PAD = NS * ROWS_PER_TILE

BLK = 2560


def _mlp_body(msg_ref, xi_ref, xj_ref, e_ref, w1a_ref, w1b_ref, w1c_ref,
              b1_ref, w2_ref, b2_ref, out_ref):
    h = jnp.dot(xj_ref[...], w1a_ref[...], preferred_element_type=jnp.float32)
    h = h + jnp.dot(e_ref[...], w1b_ref[...], preferred_element_type=jnp.float32)
    h = h + jnp.dot(xi_ref[...], w1c_ref[...], preferred_element_type=jnp.float32)
    h = h + b1_ref[...]
    h = jnp.where(h >= 0, h, 0.01 * h)
    w = jnp.sum(h * w2_ref[...], axis=1, keepdims=True) + b2_ref[0, 0]
    w = jnp.tanh(w)
    out_ref[...] = w * msg_ref[...]


def _gated_mlp(msg, x_i, x_j, e_ij, W1, b1, W2, b2):
    w1a = W1[:D]
    w1b = W1[D:D + DE]
    w1c = W1[D + DE:]
    b1r = b1.reshape(1, HIDDEN)
    w2r = W2.reshape(1, HIDDEN)
    b2r = b2.reshape(1, 1)
    grid = E // BLK
    return pl.pallas_call(
        _mlp_body,
        grid=(grid,),
        in_specs=[
            pl.BlockSpec((BLK, D), lambda i: (i, 0)),
            pl.BlockSpec((BLK, D), lambda i: (i, 0)),
            pl.BlockSpec((BLK, D), lambda i: (i, 0)),
            pl.BlockSpec((BLK, DE), lambda i: (i, 0)),
            pl.BlockSpec((D, HIDDEN), lambda i: (0, 0)),
            pl.BlockSpec((DE, HIDDEN), lambda i: (0, 0)),
            pl.BlockSpec((D, HIDDEN), lambda i: (0, 0)),
            pl.BlockSpec((1, HIDDEN), lambda i: (0, 0)),
            pl.BlockSpec((1, HIDDEN), lambda i: (0, 0)),
            pl.BlockSpec((1, 1), lambda i: (0, 0)),
        ],
        out_specs=pl.BlockSpec((BLK, D), lambda i: (i, 0)),
        out_shape=jax.ShapeDtypeStruct((E_PAD, D), jnp.float32),
    )(msg, x_i, x_j, e_ij, w1a, w1b, w1c, b1r, w2r, b2r)


def _sc_scatter_body(gated_hbm, idx_hbm, zrows_hbm, out_hbm,
                     idxbuf, gbuf, accum):
    c = lax.axis_index("c")
    s = lax.axis_index("s")
    w = c * NS + s
    pltpu.sync_copy(idx_hbm.at[pl.ds(w * CHUNKS_PER_W, CHUNKS_PER_W)], idxbuf)
    pltpu.sync_copy(zrows_hbm, accum.at[pl.ds(s * ROWS_PER_TILE, ROWS_PER_TILE)])
    plsc.subcore_barrier()

    row0 = w * EDGES_PER_W

    def chunk(j, carry):
        pltpu.sync_copy(gated_hbm.at[pl.ds(row0 + j * CHUNK, CHUNK)], gbuf)
        pltpu.sync_copy(gbuf, accum.at[idxbuf.at[j]], add=True)
        return carry

    lax.fori_loop(0, CHUNKS_PER_W, chunk, 0)
    plsc.subcore_barrier()
    pltpu.sync_copy(accum.at[pl.ds(s * ROWS_PER_TILE, ROWS_PER_TILE)],
                    out_hbm.at[c, pl.ds(s * ROWS_PER_TILE, ROWS_PER_TILE)])


_sc_scatter = functools.partial(
    pl.kernel,
    out_type=jax.ShapeDtypeStruct((NC, N_PAD, D), jnp.float32),
    mesh=plsc.VectorSubcoreMesh(core_axis_name="c", subcore_axis_name="s"),
    scratch_types=[
        pltpu.VMEM((CHUNKS_PER_W, CHUNK), jnp.int32),
        pltpu.VMEM((CHUNK, D), jnp.float32),
        pltpu.VMEM_SHARED((N_PAD, D), jnp.float32),
    ],
)(_sc_scatter_body)


CBLK = 2000


def _combine_body(a_ref, b_ref, out_ref):
    out_ref[...] = a_ref[...] + b_ref[...]


def _combine(partials):
    return pl.pallas_call(
        _combine_body,
        grid=(N // CBLK,),
        in_specs=[
            pl.BlockSpec((None, CBLK, D), lambda i: (0, i, 0)),
            pl.BlockSpec((None, CBLK, D), lambda i: (1, i, 0)),
        ],
        out_specs=pl.BlockSpec((CBLK, D), lambda i: (i, 0)),
        out_shape=jax.ShapeDtypeStruct((N, D), jnp.float32),
    )(partials, partials)


def kernel(msg, x_i, x_j, e_ij, index, num_nodes, W1, b1, W2, b2):
    gated = _gated_mlp(msg, x_i, x_j, e_ij, W1, b1, W2, b2)
    idx = index.astype(jnp.int32)
    idx_pad = jnp.concatenate(
        [idx, jnp.full((E_PAD - E,), N, jnp.int32)]).reshape(-1, CHUNK)
    zrows = jnp.zeros((ROWS_PER_TILE, D), jnp.float32)
    partials = _sc_scatter(gated, idx_pad, zrows)
    return _combine(partials)


# speedup vs baseline: 2.8664x; 1.1294x over previous
import functools

import jax
import jax.numpy as jnp
from jax import lax
from jax.experimental import pallas as pl
from jax.experimental.pallas import tpu as pltpu
import jax.experimental.pallas.tpu_sc as plsc

E = 320000
N = 10000
D = 128
DE = 16
HIDDEN = 128

NC = 2
NS = 16
CHUNK = 128
CHUNKS_PER_W = 80
EDGES_PER_W = CHUNK * CHUNKS_PER_W
E_PAD = NC * NS * EDGES_PER_W
ROWS_PER_TILE = 632
N_PAD = NS * ROWS_PER_TILE

BLK = 2560


def _mlp_body(msg_ref, xi_ref, xj_ref, e_ref, w1a_ref, w1b_ref, w1c_ref,
              b1_ref, w2_ref, b2_ref, out_ref):
    h = jnp.dot(xj_ref[...], w1a_ref[...], preferred_element_type=jnp.float32)
    h = h + jnp.dot(e_ref[...], w1b_ref[...], preferred_element_type=jnp.float32)
    h = h + jnp.dot(xi_ref[...], w1c_ref[...], preferred_element_type=jnp.float32)
    h = h + b1_ref[...]
    h = jnp.where(h >= 0, h, 0.01 * h)
    w = jnp.sum(h * w2_ref[...], axis=1, keepdims=True) + b2_ref[0, 0]
    w = jnp.tanh(w)
    out_ref[...] = w * msg_ref[...]


def _gated_mlp(msg, x_i, x_j, e_ij, W1, b1, W2, b2):
    w1a = W1[:D]
    w1b = W1[D:D + DE]
    w1c = W1[D + DE:]
    b1r = b1.reshape(1, HIDDEN)
    w2r = W2.reshape(1, HIDDEN)
    b2r = b2.reshape(1, 1)
    grid = E // BLK
    return pl.pallas_call(
        _mlp_body,
        grid=(grid,),
        in_specs=[
            pl.BlockSpec((BLK, D), lambda i: (i, 0)),
            pl.BlockSpec((BLK, D), lambda i: (i, 0)),
            pl.BlockSpec((BLK, D), lambda i: (i, 0)),
            pl.BlockSpec((BLK, DE), lambda i: (i, 0)),
            pl.BlockSpec((D, HIDDEN), lambda i: (0, 0)),
            pl.BlockSpec((DE, HIDDEN), lambda i: (0, 0)),
            pl.BlockSpec((D, HIDDEN), lambda i: (0, 0)),
            pl.BlockSpec((1, HIDDEN), lambda i: (0, 0)),
            pl.BlockSpec((1, HIDDEN), lambda i: (0, 0)),
            pl.BlockSpec((1, 1), lambda i: (0, 0)),
        ],
        out_specs=pl.BlockSpec((BLK, D), lambda i: (i, 0)),
        out_shape=jax.ShapeDtypeStruct((E_PAD, D), jnp.float32),
    )(msg, x_i, x_j, e_ij, w1a, w1b, w1c, b1r, w2r, b2r)


PAIR = CHUNK
NPAIRS = CHUNKS_PER_W


def _sc_scatter_body(gated_hbm, idx_hbm, zrows_hbm, out_hbm,
                     idxbuf, b0, b1, s0, s1, accum):
    c = lax.axis_index("c")
    s = lax.axis_index("s")
    w = c * NS + s
    pltpu.sync_copy(idx_hbm.at[pl.ds(w * CHUNKS_PER_W, CHUNKS_PER_W)], idxbuf)
    pltpu.sync_copy(zrows_hbm, accum.at[pl.ds(s * ROWS_PER_TILE, ROWS_PER_TILE)])
    plsc.subcore_barrier()

    row0 = w * EDGES_PER_W

    def start(pair, buf, sem):
        pltpu.async_copy(gated_hbm.at[pl.ds(row0 + pair * PAIR, PAIR)], buf, sem)

    def wait(pair, buf, sem):
        pltpu.make_async_copy(
            gated_hbm.at[pl.ds(row0 + pair * PAIR, PAIR)], buf, sem).wait()

    def scatter(pair, buf):
        pltpu.sync_copy(buf, accum.at[idxbuf.at[pair]], add=True)

    start(0, b0, s0)

    def body(q, carry):
        p0 = 2 * q
        start(p0 + 1, b1, s1)
        wait(p0, b0, s0)
        scatter(p0, b0)
        start(p0 + 2, b0, s0)
        wait(p0 + 1, b1, s1)
        scatter(p0 + 1, b1)
        return carry

    lax.fori_loop(0, NPAIRS // 2 - 1, body, 0)
    start(NPAIRS - 1, b1, s1)
    wait(NPAIRS - 2, b0, s0)
    scatter(NPAIRS - 2, b0)
    wait(NPAIRS - 1, b1, s1)
    scatter(NPAIRS - 1, b1)

    plsc.subcore_barrier()
    pltpu.sync_copy(accum.at[pl.ds(s * ROWS_PER_TILE, ROWS_PER_TILE)],
                    out_hbm.at[c, pl.ds(s * ROWS_PER_TILE, ROWS_PER_TILE)])


_sc_scatter = functools.partial(
    pl.kernel,
    out_type=jax.ShapeDtypeStruct((NC, N_PAD, D), jnp.float32),
    mesh=plsc.VectorSubcoreMesh(core_axis_name="c", subcore_axis_name="s"),
    scratch_types=[
        pltpu.VMEM((CHUNKS_PER_W, CHUNK), jnp.int32),
        pltpu.VMEM((PAIR, D), jnp.float32),
        pltpu.VMEM((PAIR, D), jnp.float32),
        pltpu.SemaphoreType.DMA,
        pltpu.SemaphoreType.DMA,
        pltpu.VMEM_SHARED((N_PAD, D), jnp.float32),
    ],
)(_sc_scatter_body)


CBLK = 2000


def _combine_body(a_ref, b_ref, out_ref):
    out_ref[...] = a_ref[...] + b_ref[...]


def _combine(partials):
    return pl.pallas_call(
        _combine_body,
        grid=(N // CBLK,),
        in_specs=[
            pl.BlockSpec((None, CBLK, D), lambda i: (0, i, 0)),
            pl.BlockSpec((None, CBLK, D), lambda i: (1, i, 0)),
        ],
        out_specs=pl.BlockSpec((CBLK, D), lambda i: (i, 0)),
        out_shape=jax.ShapeDtypeStruct((N, D), jnp.float32),
    )(partials, partials)


def kernel(msg, x_i, x_j, e_ij, index, num_nodes, W1, b1, W2, b2):
    gated = _gated_mlp(msg, x_i, x_j, e_ij, W1, b1, W2, b2)
    idx = index.astype(jnp.int32)
    idx_pad = jnp.concatenate(
        [idx, jnp.full((E_PAD - E,), N, jnp.int32)]).reshape(-1, CHUNK)
    zrows = jnp.zeros((ROWS_PER_TILE, D), jnp.float32)
    partials = _sc_scatter(gated, idx_pad, zrows)
    return _combine(partials)
